# Initial kernel scaffold; baseline (speedup 1.0000x reference)
import jax
import jax.numpy as jnp
from jax.experimental import pallas as pl


def kernel(hidden_states, position_ids, expert_keys, params):
    raise NotImplementedError("write your pallas kernel here")


# speedup vs baseline: 1.4200x; 1.4200x over previous
import functools

import jax
import jax.numpy as jnp
import numpy as np
from jax.experimental import pallas as pl

S, H = 2048, 1024
NH, DH = 16, 64
FF = 2048
NUM_REC = 3
NUM_EXPERTS = 2 ** NUM_REC
ROW_BLK = 512


def _pcall(body, **kw):
    return pl.pallas_call(body, **kw)


def _router_body(x_ref, ek_ref, kmask_ref, sw_ref):
    x = x_ref[...]
    scores = jax.lax.dot_general(
        x, ek_ref[...], (((1,), (1,)), ((), ())),
        preferred_element_type=jnp.float32)
    m = jnp.max(scores, axis=-1, keepdims=True)
    w = 1.0 / jnp.sum(jnp.exp(scores - m), axis=-1, keepdims=True)
    chosen = jnp.argmax(scores, axis=-1).astype(jnp.int32)
    bits = jax.lax.broadcasted_iota(jnp.int32, (S, NUM_EXPERTS), 1)
    active = ((chosen[:, None] >> bits) & 1).astype(jnp.float32)
    sw_ref[...] = active * w
    kmask_ref[...] = active.T


def _qkv_body(x_ref, ek_ref, norm_ref, pos_ref, wq_ref, wk_ref, wv_ref,
              q_ref, k_ref, v_ref, xpre_ref):
    x = x_ref[...] + ek_ref[...]
    xpre_ref[...] = x
    h = x * jax.lax.rsqrt(jnp.mean(x * x, axis=-1, keepdims=True) + 1e-6)
    h = h * norm_ref[...]
    q = jnp.dot(h, wq_ref[...], preferred_element_type=jnp.float32)
    k = jnp.dot(h, wk_ref[...], preferred_element_type=jnp.float32)
    v = jnp.dot(h, wv_ref[...], preferred_element_type=jnp.float32)

    posf = pos_ref[...].astype(jnp.float32)
    col = jax.lax.broadcasted_iota(jnp.int32, (1, H), 1)
    offs = col % DH
    f = (offs % (DH // 2)).astype(jnp.float32)
    inv = jnp.exp(f * (-np.log(10000.0) / (DH // 2)))
    ang = posf * inv
    cosf = jnp.cos(ang)
    sinf = jnp.sin(ang)
    first_half = offs < (DH // 2)

    def rope(t):
        rot_m = jnp.concatenate([t[:, DH // 2:], t[:, :DH // 2]], axis=1)
        rot_p = jnp.concatenate([t[:, -(DH // 2):], t[:, :-(DH // 2)]], axis=1)
        rot = jnp.where(first_half, -rot_m, rot_p)
        return t * cosf + rot * sinf

    q_ref[...] = rope(q)
    k_ref[...] = rope(k)
    v_ref[...] = v


def _attn_body(q_ref, k_ref, v_ref, m_ref, o_ref):
    mask = m_ref[...] > 0.5
    for sub in range(2):
        sl = slice(sub * DH, (sub + 1) * DH)
        q = q_ref[:, sl]
        att = jax.lax.dot_general(
            q, k_ref[:, sl], (((1,), (1,)), ((), ())),
            preferred_element_type=jnp.float32) * (1.0 / np.sqrt(DH))
        att = jnp.where(mask, att, -1e30)
        mx = jnp.max(att, axis=-1, keepdims=True)
        e = jnp.exp(att - mx)
        p = e / jnp.sum(e, axis=-1, keepdims=True)
        o_ref[:, sl] = jnp.dot(p, v_ref[:, sl],
                               preferred_element_type=jnp.float32)


def _post_body(final_ref, xpre_ref, o_ref, sw_ref, wo_ref, norm_ref,
               wg_ref, wu_ref, wd_ref, out_ref):
    x = xpre_ref[...] + jnp.dot(o_ref[...], wo_ref[...],
                                preferred_element_type=jnp.float32)
    h2 = x * jax.lax.rsqrt(jnp.mean(x * x, axis=-1, keepdims=True) + 1e-6)
    h2 = h2 * norm_ref[...]
    g = jnp.dot(h2, wg_ref[...], preferred_element_type=jnp.float32)
    u = jnp.dot(h2, wu_ref[...], preferred_element_type=jnp.float32)
    act = (g * jax.lax.logistic(g)) * u
    x = x + jnp.dot(act, wd_ref[...], preferred_element_type=jnp.float32)
    out_ref[...] = final_ref[...] + x * sw_ref[...]


def kernel(hidden_states, position_ids, expert_keys, params):
    Bb, Ss, Hh = hidden_states.shape
    flat = hidden_states.reshape(Ss, Hh)
    pos = position_ids.reshape(Ss, 1).astype(jnp.int32)

    kmask, sw = _pcall(
        _router_body,
        out_shape=(
            jax.ShapeDtypeStruct((NUM_EXPERTS, S), jnp.float32),
            jax.ShapeDtypeStruct((S, NUM_EXPERTS), jnp.float32),
        ),
    )(flat, expert_keys)

    nrow = S // ROW_BLK
    final = flat
    for d in range(NUM_REC):
        p = params[d]
        ek_row = expert_keys[1 << d][None, :]

        q, k, v, xpre = _pcall(
            _qkv_body,
            grid=(nrow,),
            in_specs=[
                pl.BlockSpec((ROW_BLK, H), lambda i: (i, 0)),
                pl.BlockSpec((1, H), lambda i: (0, 0)),
                pl.BlockSpec((1, H), lambda i: (0, 0)),
                pl.BlockSpec((ROW_BLK, 1), lambda i: (i, 0)),
                pl.BlockSpec((H, H), lambda i: (0, 0)),
                pl.BlockSpec((H, H), lambda i: (0, 0)),
                pl.BlockSpec((H, H), lambda i: (0, 0)),
            ],
            out_specs=[
                pl.BlockSpec((ROW_BLK, H), lambda i: (i, 0)),
                pl.BlockSpec((ROW_BLK, H), lambda i: (i, 0)),
                pl.BlockSpec((ROW_BLK, H), lambda i: (i, 0)),
                pl.BlockSpec((ROW_BLK, H), lambda i: (i, 0)),
            ],
            out_shape=[jax.ShapeDtypeStruct((S, H), jnp.float32)] * 4,
        )(final, ek_row, p["attn_norm"][None, :], pos,
          p["Wq"], p["Wk"], p["Wv"])

        o = _pcall(
            _attn_body,
            grid=(NH // 2,),
            in_specs=[
                pl.BlockSpec((S, 2 * DH), lambda h: (0, h)),
                pl.BlockSpec((S, 2 * DH), lambda h: (0, h)),
                pl.BlockSpec((S, 2 * DH), lambda h: (0, h)),
                pl.BlockSpec((1, S), lambda h: (0, 0)),
            ],
            out_specs=pl.BlockSpec((S, 2 * DH), lambda h: (0, h)),
            out_shape=jax.ShapeDtypeStruct((S, H), jnp.float32),
        )(q, k, v, kmask[d][None, :])

        final = _pcall(
            _post_body,
            grid=(nrow,),
            in_specs=[
                pl.BlockSpec((ROW_BLK, H), lambda i: (i, 0)),
                pl.BlockSpec((ROW_BLK, H), lambda i: (i, 0)),
                pl.BlockSpec((ROW_BLK, H), lambda i: (i, 0)),
                pl.BlockSpec((ROW_BLK, 1), lambda i: (i, 0)),
                pl.BlockSpec((H, H), lambda i: (0, 0)),
                pl.BlockSpec((1, H), lambda i: (0, 0)),
                pl.BlockSpec((H, FF), lambda i: (0, 0)),
                pl.BlockSpec((H, FF), lambda i: (0, 0)),
                pl.BlockSpec((FF, H), lambda i: (0, 0)),
            ],
            out_specs=pl.BlockSpec((ROW_BLK, H), lambda i: (i, 0)),
            out_shape=jax.ShapeDtypeStruct((S, H), jnp.float32),
        )(final, xpre, o, sw[:, d:d + 1], p["Wo"], p["mlp_norm"][None, :],
          p["Wg"], p["Wu"], p["Wd"])

    return final.reshape(Bb, Ss, Hh)
